# baseline (speedup 1.0000x reference)
import functools

import jax
import jax.numpy as jnp
import numpy as np
from jax import lax
from jax.experimental import pallas as pl
from jax.experimental.pallas import tpu as pltpu
from jax.experimental.pallas import tpu_sc as plsc

NC, NS, L = 2, 16, 16
NW = NC * NS
B, SEQ, D = 1024, 200, 128
V = 50176
BPW = B // NW
CH = 40
CPS = SEQ // CH
NCHUNK = BPW * CPS
DW = D // 4
ND = DW // L

_mesh = plsc.VectorSubcoreMesh(
    core_axis_name="c", subcore_axis_name="s", num_cores=NC, num_subcores=NS
)


@functools.partial(
    pl.kernel,
    out_type=jax.ShapeDtypeStruct((B, D), jnp.float32),
    mesh=_mesh,
    compiler_params=pltpu.CompilerParams(use_tc_tiling_on_sc=False),
    scratch_types=[
        pltpu.VMEM((NCHUNK, CH), jnp.int32),
        pltpu.VMEM((CH, DW), jnp.int32),
        pltpu.VMEM((CH, DW), jnp.int32),
        pltpu.VMEM((BPW, D), jnp.float32),
        pltpu.SemaphoreType.DMA,
        pltpu.SemaphoreType.DMA,
    ],
)
def _encode(x_hbm, table_hbm, out_hbm, idx_v, rows0, rows1, out_v, sem0, sem1):
    wid = lax.axis_index("s") * NC + lax.axis_index("c")
    rows = (rows0, rows1)
    sems = (sem0, sem1)

    pltpu.sync_copy(x_hbm.at[pl.ds(wid * NCHUNK, NCHUNK)], idx_v)

    zero8 = tuple(jnp.zeros((L,), jnp.int32) for _ in range(ND))
    zero32 = tuple(jnp.zeros((L,), jnp.int32) for _ in range(4 * ND))

    def fire(g, p):
        pltpu.async_copy(table_hbm.at[idx_v.at[g]], rows[p], sems[p])

    def wait(g, p):
        pltpu.make_async_copy(table_hbm.at[idx_v.at[g]], rows[p], sems[p]).wait()

    def reduce_chunk(buf, acc32):
        def row_body(r, a):
            return tuple(a[h] + buf[r, pl.ds(h * L, L)] for h in range(ND))

        acc8 = lax.fori_loop(0, CH, row_body, zero8)
        out = list(acc32)
        for h in range(ND):
            for k in range(4):
                out[4 * h + k] = out[4 * h + k] + ((acc8[h] >> (8 * k)) & 0xFF)
        return tuple(out)

    fire(0, 0)
    fire(1, 1)

    def pair_body(i, carry):
        for half in range(2):
            s = 2 * i + half
            acc32 = zero32
            for c in range(CPS):
                p = (half + c) % 2
                g = s * CPS + c
                wait(g, p)
                acc32 = reduce_chunk(rows[p], acc32)

                @pl.when(g + 2 < NCHUNK)
                def _():
                    fire(g + 2, p)

            for h in range(ND):
                for k in range(4):
                    sv = acc32[4 * h + k] - SEQ
                    out_v[s, pl.ds(32 * k + 16 * h, L)] = sv.astype(jnp.float32)
        return carry

    lax.fori_loop(0, BPW // 2, pair_body, 0)
    pltpu.sync_copy(out_v, out_hbm.at[pl.ds(wid * BPW, BPW)])


def kernel(x, table):
    x2 = x.reshape(B * CPS, CH).astype(jnp.int32)
    q = [(table[:, 32 * k : 32 * k + 32] + 1.0).astype(jnp.int32) for k in range(4)]
    tw = q[0] | (q[1] << 8) | (q[2] << 16) | (q[3] << 24)
    return _encode(x2, tw)


# speedup vs baseline: 1.2211x; 1.2211x over previous
import functools

import jax
import jax.numpy as jnp
import numpy as np
from jax import lax
from jax.experimental import pallas as pl
from jax.experimental.pallas import tpu as pltpu
from jax.experimental.pallas import tpu_sc as plsc

NC, NS, L = 2, 16, 16
NW = NC * NS
B, SEQ, D = 1024, 200, 128
V = 50176
BPW = B // NW
CHA, CHB = 104, 96
DW = D // 4
ND = DW // L

_mesh = plsc.VectorSubcoreMesh(
    core_axis_name="c", subcore_axis_name="s", num_cores=NC, num_subcores=NS
)


@functools.partial(
    pl.kernel,
    out_type=jax.ShapeDtypeStruct((B, D), jnp.float32),
    mesh=_mesh,
    compiler_params=pltpu.CompilerParams(use_tc_tiling_on_sc=False),
    scratch_types=[
        pltpu.VMEM((BPW, SEQ), jnp.int32),
        pltpu.VMEM((CHA, DW), jnp.int32),
        pltpu.VMEM((CHB, DW), jnp.int32),
        pltpu.VMEM((BPW, D), jnp.float32),
        pltpu.SemaphoreType.DMA,
        pltpu.SemaphoreType.DMA,
    ],
)
def _encode(x_hbm, table_hbm, out_hbm, idx_v, rowsa, rowsb, out_v, sema, semb):
    wid = lax.axis_index("s") * NC + lax.axis_index("c")

    pltpu.sync_copy(x_hbm.at[pl.ds(wid * BPW, BPW)], idx_v)

    zero8 = tuple(jnp.zeros((L,), jnp.int32) for _ in range(ND))
    zero32 = tuple(jnp.zeros((L,), jnp.int32) for _ in range(4 * ND))

    def fire_a(s):
        pltpu.async_copy(table_hbm.at[idx_v.at[s, pl.ds(0, CHA)]], rowsa, sema)

    def fire_b(s):
        pltpu.async_copy(table_hbm.at[idx_v.at[s, pl.ds(CHA, CHB)]], rowsb, semb)

    def wait_a(s):
        pltpu.make_async_copy(
            table_hbm.at[idx_v.at[s, pl.ds(0, CHA)]], rowsa, sema).wait()

    def wait_b(s):
        pltpu.make_async_copy(
            table_hbm.at[idx_v.at[s, pl.ds(CHA, CHB)]], rowsb, semb).wait()

    def reduce_chunk(buf, n, acc32):
        def row_body(r, a):
            return tuple(a[h] + buf[r, pl.ds(h * L, L)] for h in range(ND))

        acc8 = lax.fori_loop(0, n, row_body, zero8)
        out = list(acc32)
        for h in range(ND):
            for k in range(4):
                out[4 * h + k] = out[4 * h + k] + ((acc8[h] >> (8 * k)) & 0xFF)
        return tuple(out)

    fire_a(0)
    fire_b(0)

    def sample_body(s, carry):
        wait_a(s)
        acc32 = reduce_chunk(rowsa, CHA, zero32)

        @pl.when(s + 1 < BPW)
        def _():
            fire_a(s + 1)

        wait_b(s)
        acc32 = reduce_chunk(rowsb, CHB, acc32)

        @pl.when(s + 1 < BPW)
        def _():
            fire_b(s + 1)

        for h in range(ND):
            for k in range(4):
                sv = acc32[4 * h + k] - SEQ
                out_v[s, pl.ds(32 * k + 16 * h, L)] = sv.astype(jnp.float32)
        return carry

    lax.fori_loop(0, BPW, sample_body, 0)
    pltpu.sync_copy(out_v, out_hbm.at[pl.ds(wid * BPW, BPW)])


def kernel(x, table):
    x2 = x.astype(jnp.int32)
    q = [(table[:, 32 * k : 32 * k + 32] + 1.0).astype(jnp.int32) for k in range(4)]
    tw = q[0] | (q[1] << 8) | (q[2] << 16) | (q[3] << 24)
    return _encode(x2, tw)
